# Initial kernel scaffold; baseline (speedup 1.0000x reference)
import jax
import jax.numpy as jnp
from jax.experimental import pallas as pl


def kernel(id, text_table, id_table):
    raise NotImplementedError("write your pallas kernel here")


# speedup vs baseline: 1.2661x; 1.2661x over previous
import functools

import jax
import jax.numpy as jnp
from jax import lax
from jax.experimental import pallas as pl
from jax.experimental.pallas import tpu as pltpu
from jax.experimental.pallas import tpu_sc as plsc

NC, NS = 2, 16
NW = NC * NS
BATCH = 128
TOTAL = 1024 * 50
NBATCH = TOTAL // BATCH
GMAX = -(-NBATCH // NW)
D = 128
DN = 32

_mesh = plsc.VectorSubcoreMesh(
    core_axis_name="c", subcore_axis_name="s", num_cores=NC, num_subcores=NS)


@functools.partial(
    pl.kernel,
    out_type=jax.ShapeDtypeStruct((TOTAL, D), jnp.float32),
    mesh=_mesh,
    scratch_types=[
        pltpu.VMEM((BATCH,), jnp.int32),
        pltpu.VMEM((BATCH, D), jnp.float32),
        pltpu.VMEM((BATCH, DN), jnp.float32),
        pltpu.SemaphoreType.DMA,
        pltpu.SemaphoreType.DMA,
    ],
    compiler_params=pltpu.CompilerParams(use_tc_tiling_on_sc=False),
)
def _fused_lookup(ids_hbm, text_hbm, id_hbm, out_hbm,
                  idx_v, trow_v, irow_v, sem_t, sem_i):
    wid = lax.axis_index("s") * NC + lax.axis_index("c")
    for g in range(GMAX):
        bid = g * NW + wid

        @pl.when(bid < NBATCH)
        def _():
            base = bid * BATCH
            pltpu.sync_copy(ids_hbm.at[pl.ds(base, BATCH)], idx_v)
            ct = pltpu.async_copy(text_hbm.at[idx_v], trow_v, sem_t)
            ci = pltpu.async_copy(id_hbm.at[idx_v], irow_v, sem_i)
            ct.wait()
            ci.wait()

            def row(r, carry):
                trow_v[r, pl.ds(D - DN, 16)] += irow_v[r, pl.ds(0, 16)]
                trow_v[r, pl.ds(D - 16, 16)] += irow_v[r, pl.ds(16, 16)]
                return carry

            lax.fori_loop(0, BATCH, row, 0)
            pltpu.sync_copy(trow_v, out_hbm.at[pl.ds(base, BATCH)])


def kernel(id, text_table, id_table):
    out = _fused_lookup(id.reshape(-1), text_table, id_table)
    return out.reshape(id.shape + (D,))


# speedup vs baseline: 1.4715x; 1.1623x over previous
import functools

import jax
import jax.numpy as jnp
from jax import lax
from jax.experimental import pallas as pl
from jax.experimental.pallas import tpu as pltpu
from jax.experimental.pallas import tpu_sc as plsc

NC, NS = 2, 16
NW = NC * NS
BATCH = 128
TOTAL = 1024 * 50
NBATCH = TOTAL // BATCH
GMAX = -(-NBATCH // NW)
D = 128
DN = 32

_mesh = plsc.VectorSubcoreMesh(
    core_axis_name="c", subcore_axis_name="s", num_cores=NC, num_subcores=NS)


@functools.partial(
    pl.kernel,
    out_type=jax.ShapeDtypeStruct((TOTAL, D), jnp.float32),
    mesh=_mesh,
    scratch_types=[
        [pltpu.VMEM((BATCH,), jnp.int32)] * 2,
        [pltpu.VMEM((BATCH, D), jnp.float32)] * 2,
        [pltpu.VMEM((BATCH, DN), jnp.float32)] * 2,
        [pltpu.SemaphoreType.DMA] * 2,
        [pltpu.SemaphoreType.DMA] * 2,
        [pltpu.SemaphoreType.DMA] * 2,
    ],
    compiler_params=pltpu.CompilerParams(use_tc_tiling_on_sc=False),
)
def _fused_lookup(ids_hbm, text_hbm, id_hbm, out_hbm,
                  idx_v, trow_v, irow_v, sem_t, sem_i, sem_o):
    wid = lax.axis_index("s") * NC + lax.axis_index("c")

    def guarded(g, fn):
        if (g + 1) * NW <= NBATCH:
            fn()
        else:
            pl.when(g * NW + wid < NBATCH)(fn)

    def start_fetch(g, b):
        def _():
            base = (g * NW + wid) * BATCH
            pltpu.sync_copy(ids_hbm.at[pl.ds(base, BATCH)], idx_v[b])
            pltpu.async_copy(text_hbm.at[idx_v[b]], trow_v[b], sem_t[b])
            pltpu.async_copy(id_hbm.at[idx_v[b]], irow_v[b], sem_i[b])
        guarded(g, _)

    def wait_fetch(g, b):
        def _():
            pltpu.make_async_copy(text_hbm.at[idx_v[b]], trow_v[b], sem_t[b]).wait()
            pltpu.make_async_copy(id_hbm.at[idx_v[b]], irow_v[b], sem_i[b]).wait()
        guarded(g, _)

    def out_slice(g):
        return out_hbm.at[pl.ds((g * NW + wid) * BATCH, BATCH)]

    def wait_scatter(g, b):
        guarded(g, lambda: pltpu.make_async_copy(trow_v[b], out_slice(g), sem_o[b]).wait())

    start_fetch(0, 0)
    for g in range(GMAX):
        b = g & 1
        if g + 1 < GMAX:
            if g >= 1:
                wait_scatter(g - 1, b ^ 1)
            start_fetch(g + 1, b ^ 1)
        wait_fetch(g, b)

        def compute(b=b):
            @plsc.parallel_loop(0, BATCH, 1, unroll=8)
            def _row(r):
                trow_v[b][r, pl.ds(D - DN, 16)] += irow_v[b][r, pl.ds(0, 16)]
                trow_v[b][r, pl.ds(D - 16, 16)] += irow_v[b][r, pl.ds(16, 16)]
            pltpu.async_copy(trow_v[b], out_slice(g), sem_o[b])
        guarded(g, functools.partial(compute))

    wait_scatter(GMAX - 2, (GMAX - 2) & 1)
    wait_scatter(GMAX - 1, (GMAX - 1) & 1)


def kernel(id, text_table, id_table):
    out = _fused_lookup(id.reshape(-1), text_table, id_table)
    return out.reshape(id.shape + (D,))


# speedup vs baseline: 2.3076x; 1.5682x over previous
import functools

import jax
import jax.numpy as jnp
from jax import lax
from jax.experimental import pallas as pl
from jax.experimental.pallas import tpu as pltpu
from jax.experimental.pallas import tpu_sc as plsc

NC, NS = 2, 16
NW = NC * NS
BATCH = 128
TOTAL = 1024 * 50
NBATCH = TOTAL // BATCH
GMAX = -(-NBATCH // NW)
D = 128
DN = 32

_mesh = plsc.VectorSubcoreMesh(
    core_axis_name="c", subcore_axis_name="s", num_cores=NC, num_subcores=NS)


@functools.partial(
    pl.kernel,
    out_type=jax.ShapeDtypeStruct((TOTAL, D), jnp.float32),
    mesh=_mesh,
    scratch_types=[
        [pltpu.VMEM((BATCH,), jnp.int32)] * 2,
        [pltpu.VMEM((BATCH, D), jnp.float32)] * 2,
        [pltpu.VMEM((BATCH, DN), jnp.float32)] * 2,
        [pltpu.SemaphoreType.DMA] * 2,
        [pltpu.SemaphoreType.DMA] * 2,
        [pltpu.SemaphoreType.DMA] * 2,
    ],
    compiler_params=pltpu.CompilerParams(use_tc_tiling_on_sc=False),
)
def _fused_lookup(ids_hbm, text_hbm, id_hbm, out_hbm,
                  idx_v, trow_v, irow_v, sem_t, sem_i, sem_o):
    wid = lax.axis_index("s") * NC + lax.axis_index("c")

    def guarded(g, fn):
        if (g + 1) * NW <= NBATCH:
            fn()
        else:
            pl.when(g * NW + wid < NBATCH)(fn)

    def start_fetch(g, b):
        def _():
            base = (g * NW + wid) * BATCH
            pltpu.sync_copy(ids_hbm.at[pl.ds(base, BATCH)], idx_v[b])
            pltpu.async_copy(text_hbm.at[idx_v[b]], trow_v[b], sem_t[b])
            pltpu.async_copy(id_hbm.at[idx_v[b]], irow_v[b], sem_i[b])
        guarded(g, _)

    def wait_fetch(g, b):
        def _():
            pltpu.make_async_copy(text_hbm.at[idx_v[b]], trow_v[b], sem_t[b]).wait()
            pltpu.make_async_copy(id_hbm.at[idx_v[b]], irow_v[b], sem_i[b]).wait()
        guarded(g, _)

    def out_slice(g):
        return out_hbm.at[pl.ds((g * NW + wid) * BATCH, BATCH)]

    def wait_scatter(g, b):
        guarded(g, lambda: pltpu.make_async_copy(trow_v[b], out_slice(g), sem_o[b]).wait())

    start_fetch(0, 0)
    for g in range(GMAX):
        b = g & 1
        if g + 1 < GMAX:
            if g >= 1:
                wait_scatter(g - 1, b ^ 1)
            start_fetch(g + 1, b ^ 1)
        wait_fetch(g, b)

        def compute(b=b):
            @plsc.parallel_loop(0, BATCH, 1, unroll=8)
            def _row(r):
                trow_v[b][r, pl.ds(D - DN, 16)] += irow_v[b][r, pl.ds(0, 16)]
                trow_v[b][r, pl.ds(D - 16, 16)] += irow_v[b][r, pl.ds(16, 16)]
            pltpu.async_copy(trow_v[b], out_slice(g), sem_o[b])
        guarded(g, functools.partial(compute))

    wait_scatter(GMAX - 2, (GMAX - 2) & 1)
    wait_scatter(GMAX - 1, (GMAX - 1) & 1)


def kernel(id, text_table, id_table):
    ids_t = id.T.reshape(-1)
    out = _fused_lookup(ids_t, text_table, id_table)
    return out.reshape(id.shape[1], id.shape[0], D).transpose(1, 0, 2)
